# baseline (speedup 1.0000x reference)
import jax
import jax.numpy as jnp
from jax.experimental import pallas as pl
from jax.experimental.pallas import tpu as pltpu

_TOK_ID = 12345
_STRENGTH = 18.0

_SEQ_BLK = 512


def _body(li_ref, x_ref, w_ref, o_ref):
    b = pl.program_id(0)
    j = pl.program_id(1)
    o_ref[...] = x_ref[...]
    li = li_ref[b]
    start = j * _SEQ_BLK

    @pl.when((li >= start) & (li < start + _SEQ_BLK))
    def _():
        w = w_ref[_TOK_ID % 8, :]
        norm = jnp.sqrt(jnp.sum(w * w))
        d = (_STRENGTH / jnp.maximum(norm, 1e-12)) * w
        r = li - start
        o_ref[pl.ds(r, 1), :] = x_ref[pl.ds(r, 1), :] + d[None, :]


def kernel(x, token_ids, last_indices, lm_head_weight):
    del token_ids
    B, S, D = x.shape
    grid = (B, S // _SEQ_BLK)
    return pl.pallas_call(
        _body,
        grid=grid,
        in_specs=[
            pl.BlockSpec((32,), memory_space=pltpu.SMEM),
            pl.BlockSpec((None, _SEQ_BLK, D), lambda b, j: (b, j, 0)),
            pl.BlockSpec((8, D), lambda b, j: (_TOK_ID // 8, 0)),
        ],
        out_specs=pl.BlockSpec((None, _SEQ_BLK, D), lambda b, j: (b, j, 0)),
        out_shape=jax.ShapeDtypeStruct((B, S, D), x.dtype),
    )(last_indices, x, lm_head_weight)


# speedup vs baseline: 1.1244x; 1.1244x over previous
import jax
import jax.numpy as jnp
from jax.experimental import pallas as pl
from jax.experimental.pallas import tpu as pltpu

_TOK_ID = 12345
_STRENGTH = 18.0

_SEQ_BLK = 2048


def _body(li_ref, x_ref, w_ref, o_ref):
    b = pl.program_id(0)
    nj = pl.num_programs(1)
    o_ref[...] = x_ref[...]
    li = li_ref[b]
    if nj == 1:
        start = 0
    else:
        start = pl.program_id(1) * _SEQ_BLK

    @pl.when((li >= start) & (li < start + _SEQ_BLK))
    def _():
        w = w_ref[_TOK_ID % 8, :]
        norm = jnp.sqrt(jnp.sum(w * w))
        d = (_STRENGTH / jnp.maximum(norm, 1e-12)) * w
        r = li - start
        o_ref[pl.ds(r, 1), :] = x_ref[pl.ds(r, 1), :] + d[None, :]


def kernel(x, token_ids, last_indices, lm_head_weight):
    del token_ids
    B, S, D = x.shape
    grid = (B, S // _SEQ_BLK)
    return pl.pallas_call(
        _body,
        grid=grid,
        in_specs=[
            pl.BlockSpec(memory_space=pltpu.SMEM),
            pl.BlockSpec((None, _SEQ_BLK, D), lambda b, j: (b, j, 0)),
            pl.BlockSpec((8, D), lambda b, j: (_TOK_ID // 8, 0)),
        ],
        out_specs=pl.BlockSpec((None, _SEQ_BLK, D), lambda b, j: (b, j, 0)),
        out_shape=jax.ShapeDtypeStruct((B, S, D), x.dtype),
        compiler_params=pltpu.CompilerParams(
            dimension_semantics=("parallel", "parallel"),
        ),
    )(last_indices, x, lm_head_weight)
